# baseline (speedup 1.0000x reference)
import jax
import jax.numpy as jnp
from jax.experimental import pallas as pl
from jax.experimental.pallas import tpu as pltpu

_NUM_BINS = 8
_LEFT = 0.0
_RIGHT = 20.0
_W = (_RIGHT - _LEFT) / _NUM_BINS


def _coeffs(delta_h):
    knots = jnp.linspace(_LEFT, _RIGHT, _NUM_BINS + 1).astype(jnp.float32)
    deltas = jax.nn.softplus(delta_h)
    h = jnp.concatenate([jnp.zeros((1,), deltas.dtype), jnp.cumsum(deltas)])
    h = h / (h[-1] + 1e-06)
    s = (h[1:] - h[:-1]) / (knots[1:] - knots[:-1] + 1e-08)
    c = jnp.concatenate([s[:1], s[1:] - s[:-1]])
    t = knots[:-1]
    C = jnp.sum(c * t)
    return jnp.concatenate([c, C[None]])


def _spline_body(p_ref, x_ref, o_ref):
    bf = jnp.bfloat16
    neg = -x_ref[...]
    xb = neg.astype(bf)
    acc = jnp.full(xb.shape, 0.0, bf) - p_ref[8].astype(bf)
    for i in range(_NUM_BINS):
        acc = acc + p_ref[i].astype(bf) * jnp.maximum(xb, bf(i * _W))
    y = jnp.clip(acc, bf(0.0), bf(1.0)).astype(jnp.float32)
    o_ref[...] = jnp.where(neg > _RIGHT, 1.0, y)


def kernel(snr_db, delta_h):
    params = _coeffs(delta_h)
    n = snr_db.shape[0]
    block = 1 << 20
    out = pl.pallas_call(
        _spline_body,
        grid=(n // block,),
        in_specs=[
            pl.BlockSpec(memory_space=pltpu.SMEM),
            pl.BlockSpec((block,), lambda i: (i,)),
        ],
        out_specs=pl.BlockSpec((block,), lambda i: (i,)),
        out_shape=jax.ShapeDtypeStruct((n,), jnp.float32),
        compiler_params=pltpu.CompilerParams(
            dimension_semantics=("parallel",),
        ),
    )(params, snr_db)
    return out


# speedup vs baseline: 9.2299x; 9.2299x over previous
import jax
import jax.numpy as jnp
from jax.experimental import pallas as pl
from jax.experimental.pallas import tpu as pltpu

_NUM_BINS = 8
_LEFT = 0.0
_RIGHT = 20.0
_W = (_RIGHT - _LEFT) / _NUM_BINS


def _coeffs(delta_h):
    knots = jnp.linspace(_LEFT, _RIGHT, _NUM_BINS + 1).astype(jnp.float32)
    deltas = jax.nn.softplus(delta_h)
    h = jnp.concatenate([jnp.zeros((1,), deltas.dtype), jnp.cumsum(deltas)])
    h = h / (h[-1] + 1e-06)
    s = (h[1:] - h[:-1]) / (knots[1:] - knots[:-1] + 1e-08)
    c = jnp.concatenate([s[:1], s[1:] - s[:-1]])
    t = knots[:-1]
    C = jnp.sum(c * t)
    return jnp.concatenate([c, C[None]])


def _spline_body(p_ref, x_ref, o_ref):
    bf = jnp.bfloat16
    neg = -x_ref[...]
    xb = neg.astype(bf)
    acc = jnp.full(xb.shape, 0.0, bf) - p_ref[8].astype(bf)
    for i in range(_NUM_BINS):
        acc = acc + p_ref[i].astype(bf) * jnp.maximum(xb, bf(i * _W))
    y = jnp.clip(acc, bf(0.0), bf(1.0)).astype(jnp.float32)
    o_ref[...] = jnp.where(neg > _RIGHT, 1.0, y)


def kernel(snr_db, delta_h):
    params = _coeffs(delta_h)
    n = snr_db.shape[0]
    cols = 128
    rows = n // cols
    block_rows = 8192
    x2 = snr_db.reshape(rows, cols)
    out = pl.pallas_call(
        _spline_body,
        grid=(rows // block_rows,),
        in_specs=[
            pl.BlockSpec(memory_space=pltpu.SMEM),
            pl.BlockSpec((block_rows, cols), lambda i: (i, 0)),
        ],
        out_specs=pl.BlockSpec((block_rows, cols), lambda i: (i, 0)),
        out_shape=jax.ShapeDtypeStruct((rows, cols), jnp.float32),
        compiler_params=pltpu.CompilerParams(
            dimension_semantics=("parallel",),
        ),
    )(params, x2)
    return out.reshape(n)


# speedup vs baseline: 9.5382x; 1.0334x over previous
import jax
import jax.numpy as jnp
from jax.experimental import pallas as pl
from jax.experimental.pallas import tpu as pltpu

_NUM_BINS = 8
_LEFT = 0.0
_RIGHT = 20.0
_W = (_RIGHT - _LEFT) / _NUM_BINS


def _coeffs(delta_h):
    knots = jnp.linspace(_LEFT, _RIGHT, _NUM_BINS + 1).astype(jnp.float32)
    deltas = jax.nn.softplus(delta_h)
    h = jnp.concatenate([jnp.zeros((1,), deltas.dtype), jnp.cumsum(deltas)])
    h = h / (h[-1] + 1e-06)
    s = (h[1:] - h[:-1]) / (knots[1:] - knots[:-1] + 1e-08)
    c = jnp.concatenate([s[:1], s[1:] - s[:-1]])
    t = knots[:-1]
    C = jnp.sum(c * t)
    return jnp.concatenate([-c, C[None]])


def _spline_body(p_ref, x_ref, o_ref):
    bf = jnp.bfloat16
    sb = x_ref[...].astype(bf)
    acc = jnp.full(sb.shape, 0.0, bf) - p_ref[8].astype(bf)
    for i in range(_NUM_BINS):
        acc = acc + p_ref[i].astype(bf) * jnp.minimum(sb, bf(-i * _W))
    y = jnp.clip(acc, bf(0.0), bf(1.0))
    y = jnp.where(sb < bf(-_RIGHT), bf(1.0), y)
    o_ref[...] = y.astype(jnp.float32)


def kernel(snr_db, delta_h):
    params = _coeffs(delta_h)
    n = snr_db.shape[0]
    cols = 128
    rows = n // cols
    block_rows = 8192
    x2 = snr_db.reshape(rows, cols)
    out = pl.pallas_call(
        _spline_body,
        grid=(rows // block_rows,),
        in_specs=[
            pl.BlockSpec(memory_space=pltpu.SMEM),
            pl.BlockSpec((block_rows, cols), lambda i: (i, 0)),
        ],
        out_specs=pl.BlockSpec((block_rows, cols), lambda i: (i, 0)),
        out_shape=jax.ShapeDtypeStruct((rows, cols), jnp.float32),
        compiler_params=pltpu.CompilerParams(
            dimension_semantics=("parallel",),
        ),
    )(params, x2)
    return out.reshape(n)
